# baseline (speedup 1.0000x reference)
import functools

import jax
import jax.numpy as jnp
from jax import lax
from jax.experimental import pallas as pl
from jax.experimental.pallas import tpu as pltpu
from jax.experimental.pallas import tpu_sc as plsc

NC = 2
NS = 16
NW = NC * NS
D = 32
V = 1000000
RPAD = 1000064
NBLK = RPAD // 128
SLOTS = -(-NBLK // NW)
GRP = 128

_params = pltpu.CompilerParams(needs_layout_passes=False,
                               disable_bounds_checks=True)


@jax.jit
def _relayout(w_t):
    mesh = plsc.VectorSubcoreMesh(core_axis_name="c", subcore_axis_name="s")

    @functools.partial(
        pl.kernel,
        out_type=jax.ShapeDtypeStruct((RPAD // 4, 128), jnp.float32),
        mesh=mesh,
        scratch_types=[
            pltpu.VMEM((2, D, 128), jnp.float32),
            pltpu.VMEM((2, D, 128), jnp.float32),
            pltpu.SemaphoreType.DMA,
            pltpu.SemaphoreType.DMA,
            pltpu.SemaphoreType.DMA,
            pltpu.SemaphoreType.DMA,
        ],
        compiler_params=_params,
    )
    def k(wt_hbm, tbl_hbm, in_v, out_v, isem0, isem1, osem0, osem1):
        wid = lax.axis_index("s") * NC + lax.axis_index("c")
        lane = lax.iota(jnp.int32, 16)
        isems = (isem0, isem1)
        osems = (osem0, osem1)
        n_uniform = NBLK // NW

        def fire_in(s, buf):
            blk = wid + s * NW
            r0 = blk * 128
            for ci in range(4):
                pltpu.async_copy(
                    wt_hbm.at[pl.ds(ci * 8, 8), pl.ds(r0, 128)],
                    in_v.at[buf, pl.ds(ci * 8, 8)], isems[buf])

        def wait_in(buf):
            for ci in range(4):
                pltpu.make_async_copy(
                    wt_hbm.at[pl.ds(0, 8), pl.ds(0, 128)],
                    in_v.at[buf, pl.ds(0, 8)], isems[buf]).wait()

        def transpose(buf):
            inb = in_v.at[buf]

            @plsc.parallel_loop(0, 32, unroll=4)
            def tloop(t):
                for m in range(8):
                    rows = lane + 16 * (m % 2)
                    col = jnp.full((16,), m // 2, jnp.int32) + 4 * t
                    out_v[buf, t, pl.ds(16 * m, 16)] = plsc.load_gather(
                        inb, [rows, col])

        def fire_out(s, buf):
            blk = wid + s * NW
            pltpu.async_copy(out_v.at[buf], tbl_hbm.at[pl.ds(blk * 32, 32)],
                             osems[buf])

        def wait_out(buf):
            pltpu.make_async_copy(out_v.at[buf],
                                  tbl_hbm.at[pl.ds(0, 32)], osems[buf]).wait()

        fire_in(0, 0)

        @pl.loop(0, n_uniform, step=2)
        def pair(s):
            fire_in(s + 1, 1)
            wait_in(0)

            @pl.when(s >= 2)
            def _():
                wait_out(0)
            transpose(0)
            fire_out(s, 0)

            @pl.when(s + 2 < n_uniform)
            def _():
                fire_in(s + 2, 0)
            wait_in(1)

            @pl.when(s >= 1)
            def _():
                wait_out(1)
            transpose(1)
            fire_out(s + 1, 1)

        wait_out(0)
        wait_out(1)

        @pl.when(wid < NBLK % NW)
        def tail():
            fire_in(n_uniform, 0)
            wait_in(0)
            transpose(0)
            fire_out(n_uniform, 0)
            wait_out(0)

    return k(w_t)


@jax.jit
def _gather(idx_flat, tbl_lin):
    b_total = idx_flat.shape[0]
    b_per_w = b_total // NW
    n_groups = b_per_w // GRP
    mesh = plsc.VectorSubcoreMesh(core_axis_name="c", subcore_axis_name="s")

    @functools.partial(
        pl.kernel,
        out_type=jax.ShapeDtypeStruct((20 * 4 * 128, 8, 128), jnp.float32),
        mesh=mesh,
        scratch_types=[
            pltpu.VMEM((b_per_w,), jnp.int32),
            pltpu.VMEM((2, GRP, D), jnp.float32),
            pltpu.VMEM((D, GRP), jnp.float32),
            pltpu.SemaphoreType.DMA,
            pltpu.SemaphoreType.DMA,
        ],
        compiler_params=pltpu.CompilerParams(
            needs_layout_passes=False, disable_bounds_checks=True,
            use_tc_tiling_on_sc=False),
    )
    def k(idx_hbm, tbl_hbm, out_hbm, idx_v, rows_v, out_t, gsem0, gsem1):
        wid = lax.axis_index("s") * NC + lax.axis_index("c")
        base = wid * b_per_w
        lane = lax.iota(jnp.int32, 16)
        pltpu.sync_copy(idx_hbm.at[pl.ds(base, b_per_w)], idx_v)

        gsems = (gsem0, gsem1)

        def fire(g, buf):
            pltpu.async_copy(
                tbl_hbm.at[idx_v.at[pl.ds(g * GRP, GRP)]],
                rows_v.at[buf], gsems[buf])

        def process(g, buf):
            pltpu.make_async_copy(
                tbl_hbm.at[idx_v.at[pl.ds(0, GRP)]],
                rows_v.at[buf], gsems[buf]).wait()
            rows_g = rows_v.at[buf]

            @plsc.parallel_loop(0, GRP // 16, unroll=2)
            def mloop(m):
                ivec = lane + 16 * m
                for c in range(D):
                    col = jnp.full((16,), c, jnp.int32)
                    out_t[c, pl.ds(16 * m, 16)] = plsc.load_gather(
                        rows_g, [ivec, col])

            k0 = base + g * GRP
            s_idx = k0 // 16384
            bh = (k0 % 16384) // 128
            for ch in range(4):
                pltpu.sync_copy(
                    out_t.at[pl.ds(8 * ch, 8)],
                    out_hbm.at[s_idx * 512 + ch * 128 + bh])

        fire(0, 0)

        @pl.loop(0, n_groups, step=2)
        def pair(g):
            fire(g + 1, 1)
            process(g, 0)

            @pl.when(g + 2 < n_groups)
            def _():
                fire(g + 2, 0)
            process(g + 1, 1)

    return k(idx_flat, tbl_lin)


def kernel(token_ids, weight):
    idx_flat = token_ids.T.reshape(-1)
    w_t = weight.T
    tbl = _relayout(w_t)
    tbl_lin = tbl.reshape(RPAD, D)
    out5 = _gather(idx_flat, tbl_lin)
    t6 = out5.reshape(20, 4, 128, 8, 128)
    v = t6.transpose(0, 1, 3, 2, 4).reshape(20, D, 16384)
    return jnp.transpose(v, (2, 0, 1))


# speedup vs baseline: 1.0700x; 1.0700x over previous
import functools

import jax
import jax.numpy as jnp
from jax import lax
from jax.experimental import pallas as pl
from jax.experimental.pallas import tpu as pltpu
from jax.experimental.pallas import tpu_sc as plsc

NC = 2
NS = 16
NW = NC * NS
D = 32
V = 1000000
RPAD = 1000064
NBLK = RPAD // 128
SLOTS = -(-NBLK // NW)
GRP = 512

_params = pltpu.CompilerParams(needs_layout_passes=False,
                               disable_bounds_checks=True)


@jax.jit
def _relayout(w_t):
    mesh = plsc.VectorSubcoreMesh(core_axis_name="c", subcore_axis_name="s")

    @functools.partial(
        pl.kernel,
        out_type=jax.ShapeDtypeStruct((RPAD // 4, 128), jnp.float32),
        mesh=mesh,
        scratch_types=[
            pltpu.VMEM((4, D, 128), jnp.float32),
            pltpu.VMEM((2, D, 128), jnp.float32),
            pltpu.SemaphoreType.DMA,
            pltpu.SemaphoreType.DMA,
            pltpu.SemaphoreType.DMA,
            pltpu.SemaphoreType.DMA,
            pltpu.SemaphoreType.DMA,
            pltpu.SemaphoreType.DMA,
        ],
        compiler_params=_params,
    )
    def k(wt_hbm, tbl_hbm, in_v, out_v,
          isem0, isem1, isem2, isem3, osem0, osem1):
        wid = lax.axis_index("s") * NC + lax.axis_index("c")
        lane = lax.iota(jnp.int32, 16)
        isems = (isem0, isem1, isem2, isem3)
        osems = (osem0, osem1)
        n_uniform = NBLK // NW

        def fire_in(s, buf):
            blk = wid + s * NW
            pltpu.async_copy(
                wt_hbm.at[:, pl.ds(blk * 128, 128)],
                in_v.at[buf], isems[buf])

        def wait_in(buf):
            pltpu.make_async_copy(
                wt_hbm.at[:, pl.ds(0, 128)],
                in_v.at[buf], isems[buf]).wait()

        def transpose(buf, obuf):
            inb = in_v.at[buf]

            @plsc.parallel_loop(0, 32, unroll=4)
            def tloop(t):
                for m in range(8):
                    rows = lane + 16 * (m % 2)
                    col = jnp.full((16,), m // 2, jnp.int32) + 4 * t
                    out_v[obuf, t, pl.ds(16 * m, 16)] = plsc.load_gather(
                        inb, [rows, col])

        def fire_out(s, obuf):
            blk = wid + s * NW
            pltpu.async_copy(out_v.at[obuf], tbl_hbm.at[pl.ds(blk * 32, 32)],
                             osems[obuf])

        def wait_out(obuf):
            pltpu.make_async_copy(out_v.at[obuf],
                                  tbl_hbm.at[pl.ds(0, 32)],
                                  osems[obuf]).wait()

        for j in range(3):
            fire_in(j, j)

        @pl.loop(0, n_uniform, step=4)
        def quad(s):
            for j in range(4):
                wait_in(j)

                @pl.when(s + j >= 2)
                def _():
                    wait_out(j % 2)
                transpose(j, j % 2)
                fire_out(s + j, j % 2)

                @pl.when(s + j + 3 < n_uniform)
                def _():
                    fire_in(s + j + 3, (j + 3) % 4)

        wait_out(0)
        wait_out(1)

        @pl.when(wid < NBLK % NW)
        def tail():
            fire_in(n_uniform, 0)
            wait_in(0)
            transpose(0, 0)
            fire_out(n_uniform, 0)
            wait_out(0)

    return k(w_t)


@jax.jit
def _gather(idx_flat, tbl_lin):
    b_total = idx_flat.shape[0]
    b_per_w = b_total // NW
    n_groups = b_per_w // GRP
    mesh = plsc.VectorSubcoreMesh(core_axis_name="c", subcore_axis_name="s")

    @functools.partial(
        pl.kernel,
        out_type=jax.ShapeDtypeStruct((20 * 4 * 128, 8, 128), jnp.float32),
        mesh=mesh,
        scratch_types=[
            pltpu.VMEM((b_per_w,), jnp.int32),
            pltpu.VMEM((2, GRP, D), jnp.float32),
            pltpu.VMEM((2, GRP // 128, D, 128), jnp.float32),
            pltpu.SemaphoreType.DMA,
            pltpu.SemaphoreType.DMA,
            pltpu.SemaphoreType.DMA,
            pltpu.SemaphoreType.DMA,
        ],
        compiler_params=pltpu.CompilerParams(
            needs_layout_passes=False, disable_bounds_checks=True,
            use_tc_tiling_on_sc=False),
    )
    def k(idx_hbm, tbl_hbm, out_hbm, idx_v, rows_v, out_t,
          gsem0, gsem1, osem0, osem1):
        wid = lax.axis_index("s") * NC + lax.axis_index("c")
        base = wid * b_per_w
        lane = lax.iota(jnp.int32, 16)
        nbh = GRP // 128
        pltpu.sync_copy(idx_hbm.at[pl.ds(base, b_per_w)], idx_v)

        gsems = (gsem0, gsem1)
        osems = (osem0, osem1)

        def fire(g, buf):
            pltpu.async_copy(
                tbl_hbm.at[idx_v.at[pl.ds(g * GRP, GRP)]],
                rows_v.at[buf], gsems[buf])

        def wait_out(buf):
            for ch in range(4):
                pltpu.make_async_copy(
                    out_t.at[buf, :, pl.ds(8 * ch, 8)],
                    out_hbm.at[pl.ds(0, nbh)], osems[buf]).wait()

        def process(g, buf):
            pltpu.make_async_copy(
                tbl_hbm.at[idx_v.at[pl.ds(0, GRP)]],
                rows_v.at[buf], gsems[buf]).wait()
            rows_g = rows_v.at[buf]

            @plsc.parallel_loop(0, GRP // 16, unroll=2)
            def mloop(m):
                ivec = lane + 16 * m
                bh = m // 8
                off = 16 * (m % 8)
                for c in range(D):
                    col = jnp.full((16,), c, jnp.int32)
                    out_t[buf, bh, c, pl.ds(off, 16)] = plsc.load_gather(
                        rows_g, [ivec, col])

            k0 = base + g * GRP
            s_idx = k0 // 16384
            bh0 = (k0 % 16384) // 128
            for ch in range(4):
                pltpu.async_copy(
                    out_t.at[buf, :, pl.ds(8 * ch, 8)],
                    out_hbm.at[pl.ds(s_idx * 512 + ch * 128 + bh0, nbh)],
                    osems[buf])

        fire(0, 0)

        @pl.loop(0, n_groups, step=2)
        def pair(g):
            fire(g + 1, 1)

            @pl.when(g >= 2)
            def _():
                wait_out(0)
            process(g, 0)

            @pl.when(g + 2 < n_groups)
            def _():
                fire(g + 2, 0)

            @pl.when(g >= 1)
            def _():
                wait_out(1)
            process(g + 1, 1)

        wait_out(0)
        wait_out(1)

    return k(idx_flat, tbl_lin)


def kernel(token_ids, weight):
    idx_flat = token_ids.T.reshape(-1)
    w_t = weight.T
    tbl = _relayout(w_t)
    tbl_lin = tbl.reshape(RPAD, D)
    out5 = _gather(idx_flat, tbl_lin)
    t6 = out5.reshape(20, 4, 128, 8, 128)
    v = t6.transpose(0, 1, 3, 2, 4).reshape(20, D, 16384)
    return jnp.transpose(v, (2, 0, 1))


# speedup vs baseline: 1.3527x; 1.2642x over previous
import functools

import jax
import jax.numpy as jnp
from jax import lax
from jax.experimental import pallas as pl
from jax.experimental.pallas import tpu as pltpu
from jax.experimental.pallas import tpu_sc as plsc

NC = 2
NS = 16
NW = NC * NS
D = 32
V = 1000000
RPAD = 1000064
NBLK = RPAD // 128
SLOTS = -(-NBLK // NW)
GRP = 512

_params = pltpu.CompilerParams(needs_layout_passes=False,
                               disable_bounds_checks=True)


@jax.jit
def _relayout(w_t):
    mesh = plsc.VectorSubcoreMesh(core_axis_name="c", subcore_axis_name="s")

    @functools.partial(
        pl.kernel,
        out_type=jax.ShapeDtypeStruct((RPAD // 4, 128), jnp.float32),
        mesh=mesh,
        scratch_types=[
            pltpu.VMEM((4, D, 129), jnp.float32),
            pltpu.VMEM((2, D, 128), jnp.float32),
            pltpu.SemaphoreType.DMA,
            pltpu.SemaphoreType.DMA,
            pltpu.SemaphoreType.DMA,
            pltpu.SemaphoreType.DMA,
            pltpu.SemaphoreType.DMA,
            pltpu.SemaphoreType.DMA,
        ],
        compiler_params=_params,
    )
    def k(wt_hbm, tbl_hbm, in_v, out_v,
          isem0, isem1, isem2, isem3, osem0, osem1):
        wid = lax.axis_index("s") * NC + lax.axis_index("c")
        lane = lax.iota(jnp.int32, 16)
        isems = (isem0, isem1, isem2, isem3)
        osems = (osem0, osem1)
        n_uniform = NBLK // NW

        def fire_in(s, buf):
            blk = wid + s * NW
            pltpu.async_copy(
                wt_hbm.at[:, pl.ds(blk * 128, 128)],
                in_v.at[buf, :, pl.ds(0, 128)], isems[buf])

        def wait_in(buf):
            pltpu.make_async_copy(
                wt_hbm.at[:, pl.ds(0, 128)],
                in_v.at[buf, :, pl.ds(0, 128)], isems[buf]).wait()

        def transpose(buf, obuf):
            inb = in_v.at[buf]

            @plsc.parallel_loop(0, 32, unroll=4)
            def tloop(t):
                for m in range(8):
                    rows = lane + 16 * (m % 2)
                    col = jnp.full((16,), m // 2, jnp.int32) + 4 * t
                    out_v[obuf, t, pl.ds(16 * m, 16)] = plsc.load_gather(
                        inb, [rows, col])

        def fire_out(s, obuf):
            blk = wid + s * NW
            pltpu.async_copy(out_v.at[obuf], tbl_hbm.at[pl.ds(blk * 32, 32)],
                             osems[obuf])

        def wait_out(obuf):
            pltpu.make_async_copy(out_v.at[obuf],
                                  tbl_hbm.at[pl.ds(0, 32)],
                                  osems[obuf]).wait()

        for j in range(3):
            fire_in(j, j)

        @pl.loop(0, n_uniform, step=4)
        def quad(s):
            for j in range(4):
                wait_in(j)

                @pl.when(s + j >= 2)
                def _():
                    wait_out(j % 2)
                transpose(j, j % 2)
                fire_out(s + j, j % 2)

                @pl.when(s + j + 3 < n_uniform)
                def _():
                    fire_in(s + j + 3, (j + 3) % 4)

        wait_out(0)
        wait_out(1)

        @pl.when(wid < NBLK % NW)
        def tail():
            fire_in(n_uniform, 0)
            wait_in(0)
            transpose(0, 0)
            fire_out(n_uniform, 0)
            wait_out(0)

    return k(w_t)


@jax.jit
def _gather(idx_flat, tbl_lin):
    b_total = idx_flat.shape[0]
    b_per_w = b_total // NW
    n_groups = b_per_w // GRP
    mesh = plsc.VectorSubcoreMesh(core_axis_name="c", subcore_axis_name="s")

    @functools.partial(
        pl.kernel,
        out_type=jax.ShapeDtypeStruct((20 * 4 * 128, 8, 128), jnp.float32),
        mesh=mesh,
        scratch_types=[
            pltpu.VMEM((b_per_w,), jnp.int32),
            pltpu.VMEM((2, GRP, D), jnp.float32),
            pltpu.VMEM((2, GRP // 128, D, 129), jnp.float32),
            pltpu.SemaphoreType.DMA,
            pltpu.SemaphoreType.DMA,
            pltpu.SemaphoreType.DMA,
            pltpu.SemaphoreType.DMA,
        ],
        compiler_params=pltpu.CompilerParams(
            needs_layout_passes=False, disable_bounds_checks=True,
            use_tc_tiling_on_sc=False),
    )
    def k(idx_hbm, tbl_hbm, out_hbm, idx_v, rows_v, out_t,
          gsem0, gsem1, osem0, osem1):
        wid = lax.axis_index("s") * NC + lax.axis_index("c")
        base = wid * b_per_w
        lane = lax.iota(jnp.int32, 16)
        nbh = GRP // 128
        pltpu.sync_copy(idx_hbm.at[pl.ds(base, b_per_w)], idx_v)

        gsems = (gsem0, gsem1)
        osems = (osem0, osem1)

        def fire(g, buf):
            pltpu.async_copy(
                tbl_hbm.at[idx_v.at[pl.ds(g * GRP, GRP)]],
                rows_v.at[buf], gsems[buf])

        def wait_out(buf):
            for ch in range(4):
                pltpu.make_async_copy(
                    out_t.at[buf, :, pl.ds(8 * ch, 8), pl.ds(0, 128)],
                    out_hbm.at[pl.ds(0, nbh)], osems[buf]).wait()

        def process(g, buf):
            pltpu.make_async_copy(
                tbl_hbm.at[idx_v.at[pl.ds(0, GRP)]],
                rows_v.at[buf], gsems[buf]).wait()
            rows_g = rows_v.at[buf]
            out_b = out_t.at[buf]

            @plsc.parallel_loop(0, GRP, unroll=4)
            def iloop(i):
                bh = jnp.full((16,), i // 128, jnp.int32)
                ipos = jnp.full((16,), i % 128, jnp.int32)
                v0 = rows_g[i, pl.ds(0, 16)]
                v1 = rows_g[i, pl.ds(16, 16)]
                plsc.store_scatter(out_b, [bh, lane, ipos], v0)
                plsc.store_scatter(out_b, [bh, lane + 16, ipos], v1)

            k0 = base + g * GRP
            s_idx = k0 // 16384
            bh0 = (k0 % 16384) // 128
            for ch in range(4):
                pltpu.async_copy(
                    out_t.at[buf, :, pl.ds(8 * ch, 8), pl.ds(0, 128)],
                    out_hbm.at[pl.ds(s_idx * 512 + ch * 128 + bh0, nbh)],
                    osems[buf])

        fire(0, 0)

        @pl.loop(0, n_groups, step=2)
        def pair(g):
            fire(g + 1, 1)

            @pl.when(g >= 2)
            def _():
                wait_out(0)
            process(g, 0)

            @pl.when(g + 2 < n_groups)
            def _():
                fire(g + 2, 0)

            @pl.when(g >= 1)
            def _():
                wait_out(1)
            process(g + 1, 1)

        wait_out(0)
        wait_out(1)

    return k(idx_flat, tbl_lin)


def kernel(token_ids, weight):
    idx_flat = token_ids.T.reshape(-1)
    w_t = weight.T
    tbl = _relayout(w_t)
    tbl_lin = tbl.reshape(RPAD, D)
    out5 = _gather(idx_flat, tbl_lin)
    t6 = out5.reshape(20, 4, 128, 8, 128)
    v = t6.transpose(0, 1, 3, 2, 4).reshape(20, D, 16384)
    return jnp.transpose(v, (2, 0, 1))
